# Initial kernel scaffold; baseline (speedup 1.0000x reference)
import jax
import jax.numpy as jnp
from jax.experimental import pallas as pl


def kernel(x, edge_index, rel_type, norm, W, gamma, beta):
    raise NotImplementedError("write your pallas kernel here")


# speedup vs baseline: 22.6938x; 22.6938x over previous
import functools

import jax
import jax.numpy as jnp
from jax import lax
from jax.experimental import pallas as pl
from jax.experimental.pallas import tpu as pltpu
from jax.experimental.pallas import tpu_sc as plsc

N = 10000
E = 320000
D = 128
R = 3

NC = 2
NS = 16
NW = NC * NS
B = 128
NB = 80
EPT = NB * B
E_PAD = NW * EPT


def _xw_body(x_ref, w_ref, out_ref):
    xb = x_ref[...]
    for r in range(R):
        out_ref[r] = jnp.dot(xb, w_ref[r], preferred_element_type=jnp.float32)


def _compute_table(x, W):
    blk = 2000
    return pl.pallas_call(
        _xw_body,
        grid=(N // blk,),
        in_specs=[
            pl.BlockSpec((blk, D), lambda i: (i, 0)),
            pl.BlockSpec((R, D, D), lambda i: (0, 0, 0)),
        ],
        out_specs=pl.BlockSpec((R, blk, D), lambda i: (0, i, 0)),
        out_shape=jax.ShapeDtypeStruct((R, N, D), jnp.float32),
    )(x, W)


def _sc_body(table, srcr, relr, normr, dstr, zeros, out,
             gidx_v, norm_v, dst_v, rows_v, acc):
    c = lax.axis_index("c")
    s = lax.axis_index("s")
    wid = s * NC + c

    pltpu.sync_copy(srcr.at[wid], gidx_v)
    pltpu.sync_copy(relr.at[wid], dst_v)
    pltpu.sync_copy(normr.at[wid], norm_v)

    @pl.when(s == 0)
    def _():
        pltpu.sync_copy(zeros, acc)

    def _gix(k, carry):
        row = k // (B // 16)
        col = (k % (B // 16)) * 16
        rv = dst_v[row, pl.ds(col, 16)]
        sl = pl.ds(k * 16, 16)
        gidx_v[sl] = rv * N + gidx_v[sl]
        return carry

    lax.fori_loop(0, EPT // 16, _gix, 0)

    pltpu.sync_copy(dstr.at[wid], dst_v)

    plsc.subcore_barrier()

    def _batch(b, carry):
        pltpu.sync_copy(table.at[gidx_v.at[pl.ds(b * B, B)]], rows_v)

        def _scale(q, cc):
            nv = norm_v[b, pl.ds(q * 16, 16)]
            for jj in range(16):
                sv = nv[jj]
                j = q * 16 + jj
                for k in range(D // 16):
                    sl = pl.ds(k * 16, 16)
                    rows_v[j, sl] = rows_v[j, sl] * sv
            return cc

        lax.fori_loop(0, B // 16, _scale, 0)

        pltpu.sync_copy(rows_v, acc.at[dst_v.at[b]], add=True)
        return carry

    lax.fori_loop(0, NB, _batch, 0)

    plsc.subcore_barrier()

    @pl.when(s == 0)
    def _():
        pltpu.sync_copy(acc, out.at[c])


_sc_kernel = functools.partial(
    pl.kernel,
    out_type=jax.ShapeDtypeStruct((NC, N, D), jnp.float32),
    mesh=plsc.VectorSubcoreMesh(
        core_axis_name="c", subcore_axis_name="s", num_cores=NC,
        num_subcores=NS),
    scratch_types=[
        pltpu.VMEM((EPT,), jnp.int32),
        pltpu.VMEM((NB, B), jnp.float32),
        pltpu.VMEM((NB, B), jnp.int32),
        pltpu.VMEM((B, D), jnp.float32),
        pltpu.VMEM_SHARED((N, D), jnp.float32),
    ],
)(_sc_body)


def _bn_body(x_ref, p_ref, g_ref, b_ref, out_ref, s1, s2):
    p = pl.program_id(0)
    i = pl.program_id(1)

    @pl.when(p == 0)
    def _():
        @pl.when(i == 0)
        def _():
            s1[...] = jnp.zeros_like(s1)
            s2[...] = jnp.zeros_like(s2)

        xb = x_ref[...]
        s1[0:1] += jnp.sum(xb, axis=0, keepdims=True)
        s2[0:1] += jnp.sum(xb * xb, axis=0, keepdims=True)

    @pl.when(p == 1)
    def _():
        xb = x_ref[...]
        mean = s1[0:1] / N
        var = s2[0:1] / N - mean * mean
        inv = lax.rsqrt(var + 1e-5)
        bn = (xb - mean) * inv * g_ref[...] + b_ref[...]
        out_ref[...] = jnp.maximum(bn + p_ref[0] + p_ref[1], 0.0)


def _bn_relu(x, partials, gamma, beta):
    blk = 2000
    return pl.pallas_call(
        _bn_body,
        grid=(2, N // blk),
        in_specs=[
            pl.BlockSpec((blk, D), lambda p, i: (i, 0)),
            pl.BlockSpec((NC, blk, D), lambda p, i: (0, i, 0)),
            pl.BlockSpec((1, D), lambda p, i: (0, 0)),
            pl.BlockSpec((1, D), lambda p, i: (0, 0)),
        ],
        out_specs=pl.BlockSpec((blk, D), lambda p, i: (i, 0)),
        out_shape=jax.ShapeDtypeStruct((N, D), jnp.float32),
        scratch_shapes=[
            pltpu.VMEM((8, D), jnp.float32),
            pltpu.VMEM((8, D), jnp.float32),
        ],
    )(x, partials, gamma.reshape(1, D), beta.reshape(1, D))


def kernel(x, edge_index, rel_type, norm, W, gamma, beta):
    table = _compute_table(x, W).reshape(R * N, D)

    pad = E_PAD - E
    ar = jnp.arange(pad, dtype=jnp.int32)
    src_p = jnp.concatenate([edge_index[0], ar % N]).reshape(NW, EPT)
    dst_p = jnp.concatenate([edge_index[1], ar % N]).reshape(NW, NB, B)
    rel_p = jnp.concatenate([rel_type, jnp.zeros((pad,), jnp.int32)]).reshape(NW, NB, B)
    norm_p = jnp.concatenate([norm, jnp.zeros((pad,), jnp.float32)]).reshape(NW, NB, B)
    zeros = jnp.zeros((N, D), jnp.float32)

    partials = _sc_kernel(table, src_p, rel_p, norm_p, dst_p, zeros)
    return _bn_relu(x, partials, gamma, beta)


# speedup vs baseline: 33.3832x; 1.4710x over previous
import functools

import jax
import jax.numpy as jnp
from jax import lax
from jax.experimental import pallas as pl
from jax.experimental.pallas import tpu as pltpu
from jax.experimental.pallas import tpu_sc as plsc

N = 10000
E = 320000
D = 128
R = 3

NC = 2
NS = 16
NW = NC * NS
B = 128
NB = 80
EPT = NB * B
E_PAD = NW * EPT


def _xw_body(x_ref, w_ref, out_ref):
    xb = x_ref[...]
    for r in range(R):
        out_ref[r] = jnp.dot(xb, w_ref[r], preferred_element_type=jnp.float32)


def _compute_table(x, W):
    blk = 2000
    return pl.pallas_call(
        _xw_body,
        grid=(N // blk,),
        in_specs=[
            pl.BlockSpec((blk, D), lambda i: (i, 0)),
            pl.BlockSpec((R, D, D), lambda i: (0, 0, 0)),
        ],
        out_specs=pl.BlockSpec((R, blk, D), lambda i: (0, i, 0)),
        out_shape=jax.ShapeDtypeStruct((R, N, D), jnp.float32),
    )(x, W)


def _sc_body(table, srcr, relr, normr, dstr, zeros, out,
             gidx_v, norm_b, dst_b, rows_v, acc, sem0, sem1):
    c = lax.axis_index("c")
    s = lax.axis_index("s")
    wid = s * NC + c

    pltpu.sync_copy(srcr.at[wid], gidx_v)
    pltpu.sync_copy(relr.at[wid], rows_v.at[0, pl.ds(0, NB)])

    @pl.when(s == 0)
    def _():
        pltpu.sync_copy(zeros, acc)

    def _gix(k, carry):
        row = k // (B // 16)
        col = (k % (B // 16)) * 16
        rv = rows_v[0, row, pl.ds(col, 16)].astype(jnp.int32)
        sl = pl.ds(k * 16, 16)
        gidx_v[sl] = rv * N + gidx_v[sl]
        return carry

    lax.fori_loop(0, EPT // 16, _gix, 0)

    plsc.subcore_barrier()

    sems = (sem0, sem1)

    def _fetch_descs(b, p):
        return (
            pltpu.make_async_copy(table.at[gidx_v.at[pl.ds(b * B, B)]],
                                  rows_v.at[p], sems[p]),
            pltpu.make_async_copy(normr.at[wid, b], norm_b.at[p], sems[p]),
            pltpu.make_async_copy(dstr.at[wid, b], dst_b.at[p], sems[p]),
        )

    def _fetch_start(b, p):
        for d in _fetch_descs(b, p):
            d.start()

    def _fetch_wait(b, p):
        for d in _fetch_descs(b, p):
            d.wait()

    def _process(b, p):
        buf = rows_v.at[p]

        def _scale(q, cc):
            nv = norm_b[p, pl.ds(q * 16, 16)]
            for jj in range(16):
                sv = nv[jj]
                j = q * 16 + jj
                for k in range(D // 16):
                    sl = pl.ds(k * 16, 16)
                    buf[j, sl] = buf[j, sl] * sv
            return cc

        lax.fori_loop(0, B // 16, _scale, 0)

        pltpu.sync_copy(buf, acc.at[dst_b.at[p]], add=True)

    _fetch_start(0, 0)

    def _pipe(i, carry):
        b0 = 2 * i
        b1 = 2 * i + 1
        _fetch_start(b1, 1)
        _fetch_wait(b0, 0)
        _process(b0, 0)

        @pl.when(b1 + 1 < NB)
        def _():
            _fetch_start(b1 + 1, 0)

        _fetch_wait(b1, 1)
        _process(b1, 1)
        return carry

    lax.fori_loop(0, NB // 2, _pipe, 0)

    plsc.subcore_barrier()

    @pl.when(s == 0)
    def _():
        pltpu.sync_copy(acc, out.at[c])


_sc_kernel = functools.partial(
    pl.kernel,
    out_type=jax.ShapeDtypeStruct((NC, N, D), jnp.float32),
    mesh=plsc.VectorSubcoreMesh(
        core_axis_name="c", subcore_axis_name="s", num_cores=NC,
        num_subcores=NS),
    scratch_types=[
        pltpu.VMEM((EPT,), jnp.int32),
        pltpu.VMEM((2, B), jnp.float32),
        pltpu.VMEM((2, B), jnp.int32),
        pltpu.VMEM((2, B, D), jnp.float32),
        pltpu.VMEM_SHARED((N, D), jnp.float32),
        pltpu.SemaphoreType.DMA,
        pltpu.SemaphoreType.DMA,
    ],
)(_sc_body)


def _bn_body(x_ref, p_ref, g_ref, b_ref, out_ref, s1, s2):
    p = pl.program_id(0)
    i = pl.program_id(1)

    @pl.when(p == 0)
    def _():
        @pl.when(i == 0)
        def _():
            s1[...] = jnp.zeros_like(s1)
            s2[...] = jnp.zeros_like(s2)

        xb = x_ref[...]
        s1[0:1] += jnp.sum(xb, axis=0, keepdims=True)
        s2[0:1] += jnp.sum(xb * xb, axis=0, keepdims=True)

    @pl.when(p == 1)
    def _():
        xb = x_ref[...]
        mean = s1[0:1] / N
        var = s2[0:1] / N - mean * mean
        inv = lax.rsqrt(var + 1e-5)
        bn = (xb - mean) * inv * g_ref[...] + b_ref[...]
        out_ref[...] = jnp.maximum(bn + p_ref[0] + p_ref[1], 0.0)


def _bn_relu(x, partials, gamma, beta):
    blk = 2000
    return pl.pallas_call(
        _bn_body,
        grid=(2, N // blk),
        in_specs=[
            pl.BlockSpec((blk, D), lambda p, i: (i, 0)),
            pl.BlockSpec((NC, blk, D), lambda p, i: (0, i, 0)),
            pl.BlockSpec((1, D), lambda p, i: (0, 0)),
            pl.BlockSpec((1, D), lambda p, i: (0, 0)),
        ],
        out_specs=pl.BlockSpec((blk, D), lambda p, i: (i, 0)),
        out_shape=jax.ShapeDtypeStruct((N, D), jnp.float32),
        scratch_shapes=[
            pltpu.VMEM((8, D), jnp.float32),
            pltpu.VMEM((8, D), jnp.float32),
        ],
    )(x, partials, gamma.reshape(1, D), beta.reshape(1, D))


def kernel(x, edge_index, rel_type, norm, W, gamma, beta):
    table = _compute_table(x, W).reshape(R * N, D)

    pad = E_PAD - E
    ar = jnp.arange(pad, dtype=jnp.int32)
    src_p = jnp.concatenate([edge_index[0], ar % N]).reshape(NW, EPT)
    dst_p = jnp.concatenate([edge_index[1], ar % N]).reshape(NW, NB, B)
    rel_p = jnp.concatenate(
        [rel_type, jnp.zeros((pad,), jnp.int32)]
    ).astype(jnp.float32).reshape(NW, NB, B)
    norm_p = jnp.concatenate([norm, jnp.zeros((pad,), jnp.float32)]).reshape(NW, NB, B)
    zeros = jnp.zeros((N, D), jnp.float32)

    partials = _sc_kernel(table, src_p, rel_p, norm_p, dst_p, zeros)
    return _bn_relu(x, partials, gamma, beta)
